# Initial kernel scaffold; baseline (speedup 1.0000x reference)
import jax
import jax.numpy as jnp
from jax.experimental import pallas as pl


def kernel(x, ind, mapping, weight):
    raise NotImplementedError("write your pallas kernel here")


# speedup vs baseline: 1.1290x; 1.1290x over previous
import functools

import jax
import jax.numpy as jnp
from jax import lax
from jax.experimental import pallas as pl
from jax.experimental.pallas import tpu as pltpu
from jax.experimental.pallas import tpu_sc as plsc

_BATCH = 16384
_D = 128
_NW = 32
_BPW = _BATCH // _NW
_CHUNK = 128
_NCH = _BPW // _CHUNK


def _sc_add_gather(x, ind2, mapping, weight):
    mesh = plsc.VectorSubcoreMesh(core_axis_name="c", subcore_axis_name="s")

    @functools.partial(
        pl.kernel,
        mesh=mesh,
        out_type=jax.ShapeDtypeStruct((_BATCH, _D), jnp.float32),
        scratch_types=[
            pltpu.VMEM((_NCH, _CHUNK), jnp.int32),
            pltpu.VMEM((_CHUNK,), jnp.int32),
            pltpu.VMEM((_CHUNK, _D), jnp.float32),
            pltpu.VMEM((_CHUNK, _D), jnp.float32),
            pltpu.SemaphoreType.DMA,
        ],
    )
    def k(x_hbm, ind_hbm, map_hbm, w_hbm, out_hbm, idx_v, cid_v, rows_v, xb_v, sem):
        wid = lax.axis_index("s") * 2 + lax.axis_index("c")
        pltpu.sync_copy(ind_hbm.at[pl.ds(wid * _NCH, _NCH)], idx_v)
        for j in range(_NCH):
            r0 = wid * _BPW + j * _CHUNK
            pltpu.async_copy(map_hbm.at[idx_v.at[j]], cid_v, sem).wait()
            gcp = pltpu.async_copy(w_hbm.at[cid_v], rows_v, sem)
            pltpu.sync_copy(x_hbm.at[pl.ds(r0, _CHUNK)], xb_v)
            gcp.wait()

            def body(r, carry):
                for g in range(_D // 16):
                    s = pl.ds(g * 16, 16)
                    xb_v[r, s] = xb_v[r, s] + rows_v[r, s]
                return carry

            lax.fori_loop(0, _CHUNK, body, 0)
            pltpu.sync_copy(xb_v, out_hbm.at[pl.ds(r0, _CHUNK)])

    return k(x, ind2, mapping, weight)


def kernel(x, ind, mapping, weight):
    ind2 = ind.astype(jnp.int32).reshape(_NW * _NCH, _CHUNK)
    return _sc_add_gather(x, ind2, mapping.astype(jnp.int32), weight)


# speedup vs baseline: 1.3227x; 1.1716x over previous
import functools

import jax
import jax.numpy as jnp
from jax import lax
from jax.experimental import pallas as pl
from jax.experimental.pallas import tpu as pltpu
from jax.experimental.pallas import tpu_sc as plsc

_BATCH = 16384
_D = 128
_NW = 32
_BPW = _BATCH // _NW
_CHUNK = 128
_NCH = _BPW // _CHUNK


def _sc_add_gather(x, ind2, mapping, weight):
    mesh = plsc.VectorSubcoreMesh(core_axis_name="c", subcore_axis_name="s")

    @functools.partial(
        pl.kernel,
        mesh=mesh,
        out_type=jax.ShapeDtypeStruct((_BATCH, _D), jnp.float32),
        scratch_types=[
            pltpu.VMEM((_NCH, _CHUNK), jnp.int32),
            pltpu.VMEM((_NCH, _CHUNK), jnp.int32),
            pltpu.VMEM((2, _CHUNK, _D), jnp.float32),
            pltpu.VMEM((2, _CHUNK, _D), jnp.float32),
            pltpu.SemaphoreType.DMA,
            pltpu.SemaphoreType.DMA,
            pltpu.SemaphoreType.DMA,
            pltpu.SemaphoreType.DMA,
            pltpu.SemaphoreType.DMA,
            pltpu.SemaphoreType.DMA,
            pltpu.SemaphoreType.DMA,
        ],
    )
    def k(x_hbm, ind_hbm, map_hbm, w_hbm, out_hbm, idx_v, cid_v, rows_v, xb_v,
          csem, gsem0, gsem1, xsem0, xsem1, osem0, osem1):
        gsem = (gsem0, gsem1)
        xsem = (xsem0, xsem1)
        osem = (osem0, osem1)
        wid = lax.axis_index("s") * 2 + lax.axis_index("c")
        base = wid * _BPW
        pltpu.sync_copy(ind_hbm.at[pl.ds(wid * _NCH, _NCH)], idx_v)
        cid_cps = [
            pltpu.async_copy(map_hbm.at[idx_v.at[j]], cid_v.at[j], csem)
            for j in range(_NCH)
        ]
        for cp in cid_cps:
            cp.wait()

        def start_chunk(j):
            b = j % 2
            g = pltpu.async_copy(w_hbm.at[cid_v.at[j]], rows_v.at[b], gsem[b])
            xc = pltpu.async_copy(
                x_hbm.at[pl.ds(base + j * _CHUNK, _CHUNK)], xb_v.at[b], xsem[b])
            return g, xc

        inflight = start_chunk(0)
        out_cps = [None, None]
        for j in range(_NCH):
            b = j % 2
            nb = (j + 1) % 2
            g, xc = inflight
            if j + 1 < _NCH:
                if out_cps[nb] is not None:
                    out_cps[nb].wait()
                    out_cps[nb] = None
                inflight = start_chunk(j + 1)
            g.wait()
            xc.wait()

            def body(r, carry):
                for gi in range(_D // 16):
                    s = pl.ds(gi * 16, 16)
                    xb_v[b, r, s] = xb_v[b, r, s] + rows_v[b, r, s]
                return carry

            lax.fori_loop(0, _CHUNK, body, 0)
            out_cps[b] = pltpu.async_copy(
                xb_v.at[b], out_hbm.at[pl.ds(base + j * _CHUNK, _CHUNK)], osem[b])
        for cp in out_cps:
            if cp is not None:
                cp.wait()

    return k(x, ind2, mapping, weight)


def kernel(x, ind, mapping, weight):
    ind2 = ind.astype(jnp.int32).reshape(_NW * _NCH, _CHUNK)
    return _sc_add_gather(x, ind2, mapping.astype(jnp.int32), weight)


# speedup vs baseline: 1.3564x; 1.0255x over previous
import functools

import jax
import jax.numpy as jnp
from jax import lax
from jax.experimental import pallas as pl
from jax.experimental.pallas import tpu as pltpu
from jax.experimental.pallas import tpu_sc as plsc

_BATCH = 16384
_D = 128
_NW = 32
_BPW = _BATCH // _NW
_CHUNK = 128
_NCH = _BPW // _CHUNK


def _sc_add_gather(x, ind2, mapping, weight):
    mesh = plsc.VectorSubcoreMesh(core_axis_name="c", subcore_axis_name="s")

    @functools.partial(
        pl.kernel,
        mesh=mesh,
        out_type=jax.ShapeDtypeStruct((_BATCH, _D), jnp.float32),
        scratch_types=[
            pltpu.VMEM((_NCH, _CHUNK), jnp.int32),
            pltpu.VMEM((_NCH, _CHUNK), jnp.int32),
            pltpu.VMEM((2, _CHUNK, _D), jnp.float32),
            pltpu.VMEM((2, _CHUNK, _D), jnp.float32),
            pltpu.SemaphoreType.DMA,
            pltpu.SemaphoreType.DMA,
            pltpu.SemaphoreType.DMA,
            pltpu.SemaphoreType.DMA,
            pltpu.SemaphoreType.DMA,
            pltpu.SemaphoreType.DMA,
            pltpu.SemaphoreType.DMA,
            pltpu.SemaphoreType.DMA,
            pltpu.SemaphoreType.DMA,
            pltpu.SemaphoreType.DMA,
        ],
    )
    def k(x_hbm, ind_hbm, map_hbm, w_hbm, out_hbm, idx_v, cid_v, rows_v, xb_v,
          csem0, csem1, csem2, csem3, gsem0, gsem1, xsem0, xsem1, osem0, osem1):
        csem = (csem0, csem1, csem2, csem3)
        gsem = (gsem0, gsem1)
        xsem = (xsem0, xsem1)
        osem = (osem0, osem1)
        wid = lax.axis_index("s") * 2 + lax.axis_index("c")
        base = wid * _BPW
        x_cps = [None] * _NCH
        x_cps[0] = pltpu.async_copy(
            x_hbm.at[pl.ds(base, _CHUNK)], xb_v.at[0], xsem[0])
        pltpu.sync_copy(ind_hbm.at[pl.ds(wid * _NCH, _NCH)], idx_v)
        cid_cps = [
            pltpu.async_copy(map_hbm.at[idx_v.at[j]], cid_v.at[j], csem[j])
            for j in range(_NCH)
        ]

        g_cps = [None] * _NCH
        cid_cps[0].wait()
        g_cps[0] = pltpu.async_copy(w_hbm.at[cid_v.at[0]], rows_v.at[0], gsem[0])
        out_cps = [None, None]
        for j in range(_NCH):
            b = j % 2
            nb = (j + 1) % 2
            if j + 1 < _NCH:
                if out_cps[nb] is not None:
                    out_cps[nb].wait()
                    out_cps[nb] = None
                x_cps[j + 1] = pltpu.async_copy(
                    x_hbm.at[pl.ds(base + (j + 1) * _CHUNK, _CHUNK)],
                    xb_v.at[nb], xsem[nb])
                cid_cps[j + 1].wait()
                g_cps[j + 1] = pltpu.async_copy(
                    w_hbm.at[cid_v.at[j + 1]], rows_v.at[nb], gsem[nb])
            g_cps[j].wait()
            x_cps[j].wait()

            def body(r, carry):
                for gi in range(_D // 16):
                    s = pl.ds(gi * 16, 16)
                    xb_v[b, r, s] = xb_v[b, r, s] + rows_v[b, r, s]
                return carry

            lax.fori_loop(0, _CHUNK, body, 0)
            out_cps[b] = pltpu.async_copy(
                xb_v.at[b], out_hbm.at[pl.ds(base + j * _CHUNK, _CHUNK)], osem[b])
        for cp in out_cps:
            if cp is not None:
                cp.wait()

    return k(x, ind2, mapping, weight)


def kernel(x, ind, mapping, weight):
    ind2 = ind.astype(jnp.int32).reshape(_NW * _NCH, _CHUNK)
    return _sc_add_gather(x, ind2, mapping.astype(jnp.int32), weight)


# speedup vs baseline: 1.3648x; 1.0062x over previous
import functools

import jax
import jax.numpy as jnp
from jax import lax
from jax.experimental import pallas as pl
from jax.experimental.pallas import tpu as pltpu
from jax.experimental.pallas import tpu_sc as plsc

_BATCH = 16384
_D = 128
_NW = 32
_BPW = _BATCH // _NW
_CHUNK = 128
_NCH = _BPW // _CHUNK
_NRB = 3


def _sc_add_gather(x, ind2, mapping, weight):
    mesh = plsc.VectorSubcoreMesh(core_axis_name="c", subcore_axis_name="s")

    @functools.partial(
        pl.kernel,
        mesh=mesh,
        out_type=jax.ShapeDtypeStruct((_BATCH, _D), jnp.float32),
        scratch_types=[
            pltpu.VMEM((_NCH, _CHUNK), jnp.int32),
            pltpu.VMEM((_NCH, _CHUNK), jnp.int32),
            pltpu.VMEM((_NRB, _CHUNK, _D), jnp.float32),
            pltpu.VMEM((_NCH, _CHUNK, _D), jnp.float32),
        ]
        + [pltpu.SemaphoreType.DMA] * _NCH
        + [pltpu.SemaphoreType.DMA] * _NRB
        + [pltpu.SemaphoreType.DMA] * _NCH
        + [pltpu.SemaphoreType.DMA],
    )
    def k(x_hbm, ind_hbm, map_hbm, w_hbm, out_hbm, idx_v, cid_v, rows_v, xb_v,
          *sems):
        csem = sems[0:_NCH]
        gsem = sems[_NCH:_NCH + _NRB]
        xsem = sems[_NCH + _NRB:2 * _NCH + _NRB]
        osem = sems[2 * _NCH + _NRB]
        wid = lax.axis_index("s") * 2 + lax.axis_index("c")
        base = wid * _BPW
        x_cps = [
            pltpu.async_copy(
                x_hbm.at[pl.ds(base + j * _CHUNK, _CHUNK)], xb_v.at[j], xsem[j])
            for j in range(_NCH)
        ]
        pltpu.sync_copy(ind_hbm.at[pl.ds(wid * _NCH, _NCH)], idx_v)
        cid_cps = [
            pltpu.async_copy(map_hbm.at[idx_v.at[j]], cid_v.at[j], csem[j])
            for j in range(_NCH)
        ]

        g_cps = [None] * _NCH

        def fire_rows(j):
            cid_cps[j].wait()
            g_cps[j] = pltpu.async_copy(
                w_hbm.at[cid_v.at[j]], rows_v.at[j % _NRB], gsem[j % _NRB])

        for j in range(_NRB):
            fire_rows(j)

        out_cps = []
        for j in range(_NCH):
            rb = j % _NRB
            g_cps[j].wait()
            x_cps[j].wait()

            def body(r, carry):
                for gi in range(_D // 16):
                    s = pl.ds(gi * 16, 16)
                    xb_v[j, r, s] = xb_v[j, r, s] + rows_v[rb, r, s]
                return carry

            lax.fori_loop(0, _CHUNK, body, 0)
            out_cps.append(pltpu.async_copy(
                xb_v.at[j], out_hbm.at[pl.ds(base + j * _CHUNK, _CHUNK)], osem))
            if j + _NRB < _NCH:
                fire_rows(j + _NRB)
        for cp in out_cps:
            cp.wait()

    return k(x, ind2, mapping, weight)


def kernel(x, ind, mapping, weight):
    ind2 = ind.astype(jnp.int32).reshape(_NW * _NCH, _CHUNK)
    return _sc_add_gather(x, ind2, mapping.astype(jnp.int32), weight)


# speedup vs baseline: 1.3680x; 1.0023x over previous
import functools

import jax
import jax.numpy as jnp
from jax import lax
from jax.experimental import pallas as pl
from jax.experimental.pallas import tpu as pltpu
from jax.experimental.pallas import tpu_sc as plsc

_BATCH = 16384
_D = 128
_NW = 32
_BPW = _BATCH // _NW
_CHUNK = 128
_NCH = _BPW // _CHUNK
_NRB = 3


def _sc_add_gather(x, ind2, mapping, weight):
    mesh = plsc.VectorSubcoreMesh(core_axis_name="c", subcore_axis_name="s")

    @functools.partial(
        pl.kernel,
        mesh=mesh,
        out_type=jax.ShapeDtypeStruct((_BATCH, _D), jnp.float32),
        scratch_types=[
            pltpu.VMEM((_BPW,), jnp.int32),
            pltpu.VMEM((_NCH, _CHUNK), jnp.int32),
            pltpu.VMEM((_NRB, _CHUNK, _D), jnp.float32),
            pltpu.VMEM((_NCH, _CHUNK, _D), jnp.float32),
        ]
        + [pltpu.SemaphoreType.DMA] * _NCH
        + [pltpu.SemaphoreType.DMA] * _NRB
        + [pltpu.SemaphoreType.DMA] * _NCH
        + [pltpu.SemaphoreType.DMA],
    )
    def k(x_hbm, ind_hbm, map_hbm, w_hbm, out_hbm, idx_v, cid_v, rows_v, xb_v,
          *sems):
        csem = sems[0:_NCH]
        gsem = sems[_NCH:_NCH + _NRB]
        xsem = sems[_NCH + _NRB:2 * _NCH + _NRB]
        osem = sems[2 * _NCH + _NRB]
        wid = lax.axis_index("s") * 2 + lax.axis_index("c")
        base = wid * _BPW
        x_cps = [
            pltpu.async_copy(
                x_hbm.at[pl.ds(base + j * _CHUNK, _CHUNK)], xb_v.at[j], xsem[j])
            for j in range(_NCH)
        ]
        pltpu.sync_copy(ind_hbm.at[pl.ds(base, _BPW)], idx_v)
        cid_cps = [
            pltpu.async_copy(
                map_hbm.at[idx_v.at[pl.ds(j * _CHUNK, _CHUNK)]],
                cid_v.at[j], csem[j])
            for j in range(_NCH)
        ]

        g_cps = [None] * _NCH

        def fire_rows(j):
            cid_cps[j].wait()
            g_cps[j] = pltpu.async_copy(
                w_hbm.at[cid_v.at[j]], rows_v.at[j % _NRB], gsem[j % _NRB])

        for j in range(_NRB):
            fire_rows(j)

        out_cps = []
        for j in range(_NCH):
            rb = j % _NRB
            g_cps[j].wait()
            x_cps[j].wait()

            def body(r, carry):
                for gi in range(_D // 16):
                    s = pl.ds(gi * 16, 16)
                    xb_v[j, r, s] = xb_v[j, r, s] + rows_v[rb, r, s]
                return carry

            lax.fori_loop(0, _CHUNK, body, 0)
            out_cps.append(pltpu.async_copy(
                xb_v.at[j], out_hbm.at[pl.ds(base + j * _CHUNK, _CHUNK)], osem))
            if j + _NRB < _NCH:
                fire_rows(j + _NRB)
        for cp in out_cps:
            cp.wait()

    return k(x, ind2, mapping, weight)


def kernel(x, ind, mapping, weight):
    return _sc_add_gather(x, ind.astype(jnp.int32), mapping.astype(jnp.int32), weight)


# speedup vs baseline: 1.3875x; 1.0143x over previous
import functools

import jax
import jax.numpy as jnp
from jax import lax
from jax.experimental import pallas as pl
from jax.experimental.pallas import tpu as pltpu
from jax.experimental.pallas import tpu_sc as plsc

_BATCH = 16384
_D = 128
_NW = 32
_BPW = _BATCH // _NW
_CHUNK = 64
_NCH = _BPW // _CHUNK
_NRB = 4


def _sc_add_gather(x, ind2, mapping, weight):
    mesh = plsc.VectorSubcoreMesh(core_axis_name="c", subcore_axis_name="s")

    @functools.partial(
        pl.kernel,
        mesh=mesh,
        out_type=jax.ShapeDtypeStruct((_BATCH, _D), jnp.float32),
        scratch_types=[
            pltpu.VMEM((_BPW,), jnp.int32),
            pltpu.VMEM((_NCH, _CHUNK), jnp.int32),
            pltpu.VMEM((_NRB, _CHUNK, _D), jnp.float32),
            pltpu.VMEM((_NCH, _CHUNK, _D), jnp.float32),
        ]
        + [pltpu.SemaphoreType.DMA] * _NCH
        + [pltpu.SemaphoreType.DMA] * _NRB
        + [pltpu.SemaphoreType.DMA] * _NCH
        + [pltpu.SemaphoreType.DMA],
    )
    def k(x_hbm, ind_hbm, map_hbm, w_hbm, out_hbm, idx_v, cid_v, rows_v, xb_v,
          *sems):
        csem = sems[0:_NCH]
        gsem = sems[_NCH:_NCH + _NRB]
        xsem = sems[_NCH + _NRB:2 * _NCH + _NRB]
        osem = sems[2 * _NCH + _NRB]
        wid = lax.axis_index("s") * 2 + lax.axis_index("c")
        base = wid * _BPW
        x_cps = [
            pltpu.async_copy(
                x_hbm.at[pl.ds(base + j * _CHUNK, _CHUNK)], xb_v.at[j], xsem[j])
            for j in range(_NCH)
        ]
        pltpu.sync_copy(ind_hbm.at[pl.ds(base, _BPW)], idx_v)
        cid_cps = [
            pltpu.async_copy(
                map_hbm.at[idx_v.at[pl.ds(j * _CHUNK, _CHUNK)]],
                cid_v.at[j], csem[j])
            for j in range(_NCH)
        ]

        g_cps = [None] * _NCH

        def fire_rows(j):
            cid_cps[j].wait()
            g_cps[j] = pltpu.async_copy(
                w_hbm.at[cid_v.at[j]], rows_v.at[j % _NRB], gsem[j % _NRB])

        for j in range(_NRB):
            fire_rows(j)

        out_cps = []
        for j in range(_NCH):
            rb = j % _NRB
            g_cps[j].wait()
            x_cps[j].wait()

            def body(r, carry):
                for gi in range(_D // 16):
                    s = pl.ds(gi * 16, 16)
                    xb_v[j, r, s] = xb_v[j, r, s] + rows_v[rb, r, s]
                return carry

            lax.fori_loop(0, _CHUNK, body, 0)
            out_cps.append(pltpu.async_copy(
                xb_v.at[j], out_hbm.at[pl.ds(base + j * _CHUNK, _CHUNK)], osem))
            if j + _NRB < _NCH:
                fire_rows(j + _NRB)
        for cp in out_cps:
            cp.wait()

    return k(x, ind2, mapping, weight)


def kernel(x, ind, mapping, weight):
    return _sc_add_gather(x, ind.astype(jnp.int32), mapping.astype(jnp.int32), weight)
